# baseline (speedup 1.0000x reference)
import functools

import jax
import jax.numpy as jnp
from jax.experimental import pallas as pl
from jax.experimental.pallas import tpu as pltpu

B, S, D, H, L = 64, 512, 1024, 1024, 2

T_STEPS = 16
M_BLK = 1024


def _pre0_body(x_ref, w_ref, b_ref, o_ref):
    o_ref[...] = (
        jnp.dot(x_ref[...], w_ref[...], preferred_element_type=jnp.float32)
        + b_ref[...]
    )


def _rnn_body(pre0_ref, whh0_ref, wih1_ref, b1_ref, whh1_ref,
              out_ref, hlast_ref, h0_s, h1_s, h0blk_s, p1_s, *, n_t):
    t = pl.program_id(0)

    @pl.when(t == 0)
    def _():
        h0_s[...] = jnp.zeros_like(h0_s)
        h1_s[...] = jnp.zeros_like(h1_s)
        p1_s[1] = jnp.zeros_like(p1_s.at[1])

    parity = jax.lax.rem(t, 2)
    h0_start = h0_s[...]
    h1 = h1_s[...]
    whh0 = whh0_ref[...]
    whh1 = whh1_ref[...]

    p1_prev = p1_s.at[1 - parity]
    for tt in range(T_STEPS):
        h1 = jnp.tanh(p1_prev[tt * B:(tt + 1) * B, :] + jnp.dot(
            h1, whh1, preferred_element_type=jnp.float32))
        out_ref[:, tt, :] = h1
    h1_s[...] = h1

    h0 = h0_start
    for tt in range(T_STEPS):
        h0 = jnp.tanh(pre0_ref[:, tt, :] + jnp.dot(
            h0, whh0, preferred_element_type=jnp.float32))
        h0blk_s[tt * B:(tt + 1) * B, :] = h0
    h0_s[...] = h0
    p1_s[parity] = (
        jnp.dot(h0blk_s[...], wih1_ref[...],
                preferred_element_type=jnp.float32)
        + b1_ref[...]
    )

    @pl.when(t == n_t)
    def _():
        hlast_ref[0, :, :] = h0_start
        hlast_ref[1, :, :] = h1


def kernel(x, Wih0, b0, Whh0, Wih1, b1, Whh1):
    b0_2d = b0.reshape(1, H)
    b1_2d = b1.reshape(1, H)

    xf = x.reshape(B * S, D)
    m_tiles = (B * S) // M_BLK
    pre0 = pl.pallas_call(
        _pre0_body,
        grid=(m_tiles,),
        in_specs=[
            pl.BlockSpec((M_BLK, D), lambda j: (j, 0)),
            pl.BlockSpec((D, H), lambda j: (0, 0)),
            pl.BlockSpec((1, H), lambda j: (0, 0)),
        ],
        out_specs=pl.BlockSpec((M_BLK, H), lambda j: (j, 0)),
        out_shape=jax.ShapeDtypeStruct((B * S, H), jnp.float32),
        compiler_params=pltpu.CompilerParams(
            dimension_semantics=("arbitrary",),
        ),
        name="rnn_pre0",
    )(xf, Wih0, b0_2d).reshape(B, S, H)

    n_t = S // T_STEPS
    out, hlast = pl.pallas_call(
        functools.partial(_rnn_body, n_t=n_t),
        grid=(n_t + 1,),
        in_specs=[
            pl.BlockSpec((B, T_STEPS, H),
                         lambda t: (0, jnp.minimum(t, n_t - 1), 0)),
            pl.BlockSpec((H, H), lambda t: (0, 0)),
            pl.BlockSpec((H, H), lambda t: (0, 0)),
            pl.BlockSpec((1, H), lambda t: (0, 0)),
            pl.BlockSpec((H, H), lambda t: (0, 0)),
        ],
        out_specs=[
            pl.BlockSpec((B, T_STEPS, H),
                         lambda t: (0, jnp.maximum(t - 1, 0), 0)),
            pl.BlockSpec((L, B, H), lambda t: (0, 0, 0)),
        ],
        out_shape=[
            jax.ShapeDtypeStruct((B, S, H), jnp.float32),
            jax.ShapeDtypeStruct((L, B, H), jnp.float32),
        ],
        scratch_shapes=[
            pltpu.VMEM((B, H), jnp.float32),
            pltpu.VMEM((B, H), jnp.float32),
            pltpu.VMEM((T_STEPS * B, H), jnp.float32),
            pltpu.VMEM((2, T_STEPS * B, H), jnp.float32),
        ],
        compiler_params=pltpu.CompilerParams(
            dimension_semantics=("arbitrary",),
            vmem_limit_bytes=56 * 1024 * 1024,
        ),
        name="rnn_scan",
    )(pre0, Whh0, Wih1, b1_2d, Whh1)
    return out, hlast


# speedup vs baseline: 1.0101x; 1.0101x over previous
import functools

import jax
import jax.numpy as jnp
from jax import lax
from jax.experimental import pallas as pl
from jax.experimental.pallas import tpu as pltpu


def _mdot(a_f32, w_bf16):
    return lax.dot_general(a_f32, w_bf16, (((1,), (0,)), ((), ())),
                           preferred_element_type=jnp.float32)

B, S, D, H, L = 64, 512, 1024, 1024, 2

T_STEPS = 16
M_BLK = 1024


def _pre0_body(x_ref, w_ref, b_ref, o_ref):
    o_ref[...] = _mdot(x_ref[...], w_ref[...]) + b_ref[...]


def _rnn_body(pre0_ref, whh0_ref, wih1_ref, b1_ref, whh1_ref,
              out_ref, hlast_ref, h0_s, h1_s, h0blk_s, p1_s, *, n_t):
    t = pl.program_id(0)

    @pl.when(t == 0)
    def _():
        h0_s[...] = jnp.zeros_like(h0_s)
        h1_s[...] = jnp.zeros_like(h1_s)

    whh0 = whh0_ref[...]
    whh1 = whh1_ref[...]

    h0 = h0_s[...]
    for tt in range(T_STEPS):
        h0 = jnp.tanh(pre0_ref[:, tt, :] + _mdot(h0, whh0))
        h0blk_s[tt * B:(tt + 1) * B, :] = h0
    h0_s[...] = h0

    p1_s[...] = _mdot(h0blk_s[...], wih1_ref[...]) + b1_ref[...]

    h1 = h1_s[...]
    for tt in range(T_STEPS):
        h1 = jnp.tanh(p1_s[tt * B:(tt + 1) * B, :] + _mdot(h1, whh1))
        out_ref[:, tt, :] = h1
    h1_s[...] = h1

    @pl.when(t == n_t - 1)
    def _():
        hlast_ref[0, :, :] = h0
        hlast_ref[1, :, :] = h1


def kernel(x, Wih0, b0, Whh0, Wih1, b1, Whh1):
    b0_2d = b0.reshape(1, H)
    b1_2d = b1.reshape(1, H)
    Wih0 = Wih0.astype(jnp.bfloat16)
    Whh0 = Whh0.astype(jnp.bfloat16)
    Wih1 = Wih1.astype(jnp.bfloat16)
    Whh1 = Whh1.astype(jnp.bfloat16)

    xf = x.reshape(B * S, D)
    m_tiles = (B * S) // M_BLK
    pre0 = pl.pallas_call(
        _pre0_body,
        grid=(m_tiles,),
        in_specs=[
            pl.BlockSpec((M_BLK, D), lambda j: (j, 0)),
            pl.BlockSpec((D, H), lambda j: (0, 0)),
            pl.BlockSpec((1, H), lambda j: (0, 0)),
        ],
        out_specs=pl.BlockSpec((M_BLK, H), lambda j: (j, 0)),
        out_shape=jax.ShapeDtypeStruct((B * S, H), jnp.float32),
        compiler_params=pltpu.CompilerParams(
            dimension_semantics=("arbitrary",),
        ),
        name="rnn_pre0",
    )(xf, Wih0, b0_2d).reshape(B, S, H)

    n_t = S // T_STEPS
    out, hlast = pl.pallas_call(
        functools.partial(_rnn_body, n_t=n_t),
        grid=(n_t,),
        in_specs=[
            pl.BlockSpec((B, T_STEPS, H), lambda t: (0, t, 0)),
            pl.BlockSpec((H, H), lambda t: (0, 0)),
            pl.BlockSpec((H, H), lambda t: (0, 0)),
            pl.BlockSpec((1, H), lambda t: (0, 0)),
            pl.BlockSpec((H, H), lambda t: (0, 0)),
        ],
        out_specs=[
            pl.BlockSpec((B, T_STEPS, H), lambda t: (0, t, 0)),
            pl.BlockSpec((L, B, H), lambda t: (0, 0, 0)),
        ],
        out_shape=[
            jax.ShapeDtypeStruct((B, S, H), jnp.float32),
            jax.ShapeDtypeStruct((L, B, H), jnp.float32),
        ],
        scratch_shapes=[
            pltpu.VMEM((B, H), jnp.float32),
            pltpu.VMEM((B, H), jnp.float32),
            pltpu.VMEM((T_STEPS * B, H), jnp.float32),
            pltpu.VMEM((T_STEPS * B, H), jnp.float32),
        ],
        compiler_params=pltpu.CompilerParams(
            dimension_semantics=("arbitrary",),
            vmem_limit_bytes=56 * 1024 * 1024,
        ),
        name="rnn_scan",
    )(pre0, Whh0, Wih1, b1_2d, Whh1)
    return out, hlast


# speedup vs baseline: 1.0248x; 1.0145x over previous
import functools

import jax
import jax.numpy as jnp
from jax.experimental import pallas as pl
from jax.experimental.pallas import tpu as pltpu

B, S, D, H, L = 64, 512, 1024, 1024, 2

T_STEPS = 16
M_BLK = 1024


def _pre0_body(x_ref, w_ref, b_ref, o_ref):
    o_ref[...] = (
        jnp.dot(x_ref[...], w_ref[...], preferred_element_type=jnp.float32)
        + b_ref[...]
    )


def _rnn_body(pre0_ref, whh0_hbm, wih1_hbm, b1_ref, whh1_hbm,
              out_ref, hlast_ref, h0_s, h1_s, h0blk_s, p1_s,
              whh0_s, wih1_s, whh1_s, wsem, *, n_t):
    t = pl.program_id(0)

    @pl.when(t == 0)
    def _():
        h0_s[...] = jnp.zeros_like(h0_s)
        h1_s[...] = jnp.zeros_like(h1_s)
        c0 = pltpu.make_async_copy(whh0_hbm, whh0_s, wsem.at[0])
        c1 = pltpu.make_async_copy(wih1_hbm, wih1_s, wsem.at[1])
        c2 = pltpu.make_async_copy(whh1_hbm, whh1_s, wsem.at[2])
        c0.start()
        c1.start()
        c2.start()
        c0.wait()
        c1.wait()
        c2.wait()

    whh0 = whh0_s[...]
    whh1 = whh1_s[...]

    h0 = h0_s[...]
    for tt in range(T_STEPS):
        h0 = jnp.tanh(pre0_ref[:, tt, :] + jnp.dot(
            h0, whh0, preferred_element_type=jnp.float32))
        h0blk_s[tt * B:(tt + 1) * B, :] = h0
    h0_s[...] = h0

    p1_s[...] = (
        jnp.dot(h0blk_s[...], wih1_s[...],
                preferred_element_type=jnp.float32)
        + b1_ref[...]
    )

    h1 = h1_s[...]
    for tt in range(T_STEPS):
        h1 = jnp.tanh(p1_s[tt * B:(tt + 1) * B, :] + jnp.dot(
            h1, whh1, preferred_element_type=jnp.float32))
        out_ref[:, tt, :] = h1
    h1_s[...] = h1

    @pl.when(t == n_t - 1)
    def _():
        hlast_ref[0, :, :] = h0
        hlast_ref[1, :, :] = h1


def kernel(x, Wih0, b0, Whh0, Wih1, b1, Whh1):
    b0_2d = b0.reshape(1, H)
    b1_2d = b1.reshape(1, H)

    xf = x.reshape(B * S, D)
    m_tiles = (B * S) // M_BLK
    pre0 = pl.pallas_call(
        _pre0_body,
        grid=(m_tiles,),
        in_specs=[
            pl.BlockSpec((M_BLK, D), lambda j: (j, 0)),
            pl.BlockSpec((D, H), lambda j: (0, 0)),
            pl.BlockSpec((1, H), lambda j: (0, 0)),
        ],
        out_specs=pl.BlockSpec((M_BLK, H), lambda j: (j, 0)),
        out_shape=jax.ShapeDtypeStruct((B * S, H), jnp.float32),
        compiler_params=pltpu.CompilerParams(
            dimension_semantics=("arbitrary",),
        ),
        name="rnn_pre0",
    )(xf, Wih0, b0_2d).reshape(B, S, H)

    n_t = S // T_STEPS
    out, hlast = pl.pallas_call(
        functools.partial(_rnn_body, n_t=n_t),
        grid=(n_t,),
        in_specs=[
            pl.BlockSpec((B, T_STEPS, H), lambda t: (0, t, 0)),
            pl.BlockSpec(memory_space=pl.ANY),
            pl.BlockSpec(memory_space=pl.ANY),
            pl.BlockSpec((1, H), lambda t: (0, 0)),
            pl.BlockSpec(memory_space=pl.ANY),
        ],
        out_specs=[
            pl.BlockSpec((B, T_STEPS, H), lambda t: (0, t, 0)),
            pl.BlockSpec((L, B, H), lambda t: (0, 0, 0)),
        ],
        out_shape=[
            jax.ShapeDtypeStruct((B, S, H), jnp.float32),
            jax.ShapeDtypeStruct((L, B, H), jnp.float32),
        ],
        scratch_shapes=[
            pltpu.VMEM((B, H), jnp.float32),
            pltpu.VMEM((B, H), jnp.float32),
            pltpu.VMEM((T_STEPS * B, H), jnp.float32),
            pltpu.VMEM((T_STEPS * B, H), jnp.float32),
            pltpu.VMEM((H, H), jnp.float32),
            pltpu.VMEM((H, H), jnp.float32),
            pltpu.VMEM((H, H), jnp.float32),
            pltpu.SemaphoreType.DMA((3,)),
        ],
        compiler_params=pltpu.CompilerParams(
            dimension_semantics=("arbitrary",),
            vmem_limit_bytes=56 * 1024 * 1024,
        ),
        name="rnn_scan",
    )(pre0, Whh0, Wih1, b1_2d, Whh1)
    return out, hlast


# speedup vs baseline: 1.0796x; 1.0535x over previous
import functools

import jax
import jax.numpy as jnp
from jax.experimental import pallas as pl
from jax.experimental.pallas import tpu as pltpu

B, S, D, H, L = 64, 512, 1024, 1024, 2

T_STEPS = 16


def _rnn_body(x_ref, wih0_hbm, b0_ref, whh0_hbm, wih1_hbm, b1_ref, whh1_hbm,
              out_ref, hlast_ref, h0_s, h1_s, z_s, h0blk_s, p1_s,
              wih0_s, whh0_s, wih1_s, whh1_s, wsem, *, n_t):
    t = pl.program_id(0)

    @pl.when(t == 0)
    def _():
        h0_s[...] = jnp.zeros_like(h0_s)
        h1_s[...] = jnp.zeros_like(h1_s)
        c0 = pltpu.make_async_copy(wih0_hbm, wih0_s, wsem.at[0])
        c1 = pltpu.make_async_copy(whh0_hbm, whh0_s, wsem.at[1])
        c2 = pltpu.make_async_copy(wih1_hbm, wih1_s, wsem.at[2])
        c3 = pltpu.make_async_copy(whh1_hbm, whh1_s, wsem.at[3])
        c0.start()
        c1.start()
        c2.start()
        c3.start()
        c0.wait()
        c1.wait()
        c2.wait()
        c3.wait()

    whh0 = whh0_s[...]
    whh1 = whh1_s[...]

    z_s[...] = (
        jnp.dot(x_ref[...].reshape(B * T_STEPS, D), wih0_s[...],
                preferred_element_type=jnp.float32)
        + b0_ref[...]
    ).reshape(B, T_STEPS, H)

    h0 = h0_s[...]
    for tt in range(T_STEPS):
        h0 = jnp.tanh(z_s[:, tt, :] + jnp.dot(
            h0, whh0, preferred_element_type=jnp.float32))
        h0blk_s[tt * B:(tt + 1) * B, :] = h0
    h0_s[...] = h0

    p1_s[...] = (
        jnp.dot(h0blk_s[...], wih1_s[...],
                preferred_element_type=jnp.float32)
        + b1_ref[...]
    )

    h1 = h1_s[...]
    for tt in range(T_STEPS):
        h1 = jnp.tanh(p1_s[tt * B:(tt + 1) * B, :] + jnp.dot(
            h1, whh1, preferred_element_type=jnp.float32))
        out_ref[:, tt, :] = h1
    h1_s[...] = h1

    @pl.when(t == n_t - 1)
    def _():
        hlast_ref[0, :, :] = h0
        hlast_ref[1, :, :] = h1


def kernel(x, Wih0, b0, Whh0, Wih1, b1, Whh1):
    b0_2d = b0.reshape(1, H)
    b1_2d = b1.reshape(1, H)

    n_t = S // T_STEPS
    out, hlast = pl.pallas_call(
        functools.partial(_rnn_body, n_t=n_t),
        grid=(n_t,),
        in_specs=[
            pl.BlockSpec((B, T_STEPS, D), lambda t: (0, t, 0)),
            pl.BlockSpec(memory_space=pl.ANY),
            pl.BlockSpec((1, H), lambda t: (0, 0)),
            pl.BlockSpec(memory_space=pl.ANY),
            pl.BlockSpec(memory_space=pl.ANY),
            pl.BlockSpec((1, H), lambda t: (0, 0)),
            pl.BlockSpec(memory_space=pl.ANY),
        ],
        out_specs=[
            pl.BlockSpec((B, T_STEPS, H), lambda t: (0, t, 0)),
            pl.BlockSpec((L, B, H), lambda t: (0, 0, 0)),
        ],
        out_shape=[
            jax.ShapeDtypeStruct((B, S, H), jnp.float32),
            jax.ShapeDtypeStruct((L, B, H), jnp.float32),
        ],
        scratch_shapes=[
            pltpu.VMEM((B, H), jnp.float32),
            pltpu.VMEM((B, H), jnp.float32),
            pltpu.VMEM((B, T_STEPS, H), jnp.float32),
            pltpu.VMEM((T_STEPS * B, H), jnp.float32),
            pltpu.VMEM((T_STEPS * B, H), jnp.float32),
            pltpu.VMEM((D, H), jnp.float32),
            pltpu.VMEM((H, H), jnp.float32),
            pltpu.VMEM((H, H), jnp.float32),
            pltpu.VMEM((H, H), jnp.float32),
            pltpu.SemaphoreType.DMA((4,)),
        ],
        compiler_params=pltpu.CompilerParams(
            dimension_semantics=("arbitrary",),
            vmem_limit_bytes=56 * 1024 * 1024,
        ),
        name="rnn_scan",
    )(x, Wih0, b0_2d, Whh0, Wih1, b1_2d, Whh1)
    return out, hlast
